# baseline (speedup 1.0000x reference)
import functools
import math

import jax
import jax.numpy as jnp
from jax import lax
from jax.experimental import pallas as pl
from jax.experimental.pallas import tpu as pltpu
from jax.experimental.pallas import tpu_sc as plsc

F = 128
N_RBF = 20
R_CUT = 5.0
N_NODES = 10000


_BE_SS = 4000


def _sumsq_body(r_ref, out_ref):
    i = pl.program_id(0)

    @pl.when(i == 0)
    def _():
        out_ref[0, 0] = 0.0

    blk = r_ref[...]
    out_ref[0, 0] += jnp.sum(blk * blk)


def _sumsq(r):
    E = r.shape[0]
    return pl.pallas_call(
        _sumsq_body,
        grid=(E // _BE_SS,),
        in_specs=[pl.BlockSpec((_BE_SS, 3), lambda i: (i, 0))],
        out_specs=pl.BlockSpec((1, 1), lambda i: (0, 0),
                               memory_space=pltpu.SMEM),
        out_shape=jax.ShapeDtypeStruct((1, 1), jnp.float32),
    )(r)


_BE = 1280


def _msg_body(s_ref, r_ref, rt_ref, v_ref, wphi_ref, bphi_ref, ww_ref, bw_ref,
              ss_ref, out_ref):
    sblk = s_ref[...]
    phi = jnp.dot(sblk, wphi_ref[...],
                  preferred_element_type=jnp.float32) + bphi_ref[...]

    rblk = r_ref[...]
    rt = rt_ref[...]
    rn2t = jnp.sum(rt * rt, axis=0, keepdims=True)
    invt = lax.rsqrt(rn2t)
    rnt = rn2t * invt
    nvals = ((lax.broadcasted_iota(jnp.int32, (N_RBF, 1), 0) + 1)
             .astype(jnp.float32) * (math.pi / R_CUT))
    rbt = jnp.sin(rnt * nvals) * invt
    rbt = jnp.where(rbt <= R_CUT,
                    0.5 * (jnp.cos(rbt * (math.pi / R_CUT)) + 1.0),
                    0.0)
    w = lax.dot_general(rbt, ww_ref[...],
                        dimension_numbers=(((0,), (0,)), ((), ())),
                        preferred_element_type=jnp.float32) + bw_ref[...]

    split = w * phi
    s0 = split[:, :F]
    s1 = split[:, F:2 * F]
    s2 = split[:, 2 * F:]

    ginv = lax.rsqrt(ss_ref[0, 0])
    vblk = v_ref[...]
    out_ref[0] = s1
    for d in range(3):
        rd = rblk[:, d:d + 1] * ginv
        out_ref[d + 1] = s0 * vblk[:, d * F:(d + 1) * F] + s2 * rd


def _edge_messages(s2d, r, rt, v2d, W_phi, b_phi2, W_w, b_w2, ss):
    E = s2d.shape[0]
    return pl.pallas_call(
        _msg_body,
        grid=(E // _BE,),
        in_specs=[
            pl.BlockSpec((_BE, F), lambda i: (i, 0)),
            pl.BlockSpec((_BE, 3), lambda i: (i, 0)),
            pl.BlockSpec((3, _BE), lambda i: (0, i)),
            pl.BlockSpec((_BE, 3 * F), lambda i: (i, 0)),
            pl.BlockSpec((F, 3 * F), lambda i: (0, 0)),
            pl.BlockSpec((1, 3 * F), lambda i: (0, 0)),
            pl.BlockSpec((N_RBF, 3 * F), lambda i: (0, 0)),
            pl.BlockSpec((1, 3 * F), lambda i: (0, 0)),
            pl.BlockSpec((1, 1), lambda i: (0, 0), memory_space=pltpu.SMEM),
        ],
        out_specs=pl.BlockSpec((4, _BE, F), lambda i: (0, i, 0)),
        out_shape=jax.ShapeDtypeStruct((4, E, F), jnp.float32),
    )(s2d, r, rt, v2d, W_phi, b_phi2, W_w, b_w2, ss)


_C = 80


def _sc_segment_sum(msg, idx, zrows, E):
    n_tiles = 16
    ept = E // n_tiles
    nch = ept // _C
    ach = N_NODES // _C

    mesh = plsc.VectorSubcoreMesh(core_axis_name="c", subcore_axis_name="s")

    @functools.partial(
        pl.kernel,
        mesh=mesh,
        out_type=jax.ShapeDtypeStruct((4, N_NODES, F), jnp.float32),
        scratch_types=[
            pltpu.VMEM((_C,), jnp.int32),
            pltpu.VMEM((_C, F), jnp.float32),
            pltpu.VMEM_SHARED((N_NODES, F), jnp.float32),
        ],
    )
    def run(msg_hbm, idx_hbm, zrows_hbm, out_hbm, idx_v, rows_v, acc):
        core = lax.axis_index("c")
        sub = lax.axis_index("s")
        my_acc_chunks = (ach - sub + n_tiles - 1) // n_tiles

        for p in range(2):
            plane = core * 2 + p

            pltpu.sync_copy(zrows_hbm, rows_v)

            def zbody(k, carry):
                ci = sub + n_tiles * k
                pltpu.sync_copy(rows_v, acc.at[pl.ds(ci * _C, _C)])
                return carry

            lax.fori_loop(0, my_acc_chunks, zbody, 0)
            plsc.subcore_barrier()

            ebase = sub * ept

            def cbody(ci, carry):
                base = ebase + ci * _C
                pltpu.sync_copy(idx_hbm.at[pl.ds(base, _C)], idx_v)
                pltpu.sync_copy(msg_hbm.at[plane, pl.ds(base, _C)], rows_v)
                pltpu.sync_copy(rows_v, acc.at[idx_v], add=True)
                return carry

            lax.fori_loop(0, nch, cbody, 0)
            plsc.subcore_barrier()

            def fbody(k, carry):
                ci = sub + n_tiles * k
                pltpu.sync_copy(acc.at[pl.ds(ci * _C, _C)],
                                out_hbm.at[plane, pl.ds(ci * _C, _C)])
                return carry

            lax.fori_loop(0, my_acc_chunks, fbody, 0)
            plsc.subcore_barrier()

    return run(msg, idx, zrows)


_BN = 2000


def _unpack_body(acc_ref, outv_ref, outs_ref):
    outs_ref[:, 0, :] = acc_ref[0]
    for d in range(3):
        outv_ref[:, d, :] = acc_ref[d + 1]


def _unpack(acc4):
    return pl.pallas_call(
        _unpack_body,
        grid=(N_NODES // _BN,),
        in_specs=[pl.BlockSpec((4, _BN, F), lambda i: (0, i, 0))],
        out_specs=[
            pl.BlockSpec((_BN, 3, F), lambda i: (i, 0, 0)),
            pl.BlockSpec((_BN, 1, F), lambda i: (i, 0, 0)),
        ],
        out_shape=[
            jax.ShapeDtypeStruct((N_NODES, 3, F), jnp.float32),
            jax.ShapeDtypeStruct((N_NODES, 1, F), jnp.float32),
        ],
    )(acc4)


def kernel(s, r, v, W_phi, b_phi, W_w, b_w, idx_i):
    E = s.shape[0]
    s2d = s.reshape(E, F)
    v2d = v.reshape(E, 3 * F)
    idx = idx_i.astype(jnp.int32)
    b_phi2 = b_phi.reshape(1, 3 * F)
    b_w2 = b_w.reshape(1, 3 * F)

    rt = r.T
    ss = _sumsq(r)
    msg = _edge_messages(s2d, r, rt, v2d, W_phi, b_phi2, W_w, b_w2, ss)
    zrows = jnp.zeros((_C, F), dtype=jnp.float32)
    acc4 = _sc_segment_sum(msg, idx, zrows, E)
    out_v, out_s = _unpack(acc4)
    return (out_v, out_s)


# speedup vs baseline: 1.0752x; 1.0752x over previous
import functools
import math

import jax
import jax.numpy as jnp
from jax import lax
from jax.experimental import pallas as pl
from jax.experimental.pallas import tpu as pltpu
from jax.experimental.pallas import tpu_sc as plsc

F = 128
N_RBF = 20
R_CUT = 5.0
N_NODES = 10000


_BE_SS = 4000


def _sumsq_body(r_ref, out_ref):
    i = pl.program_id(0)

    @pl.when(i == 0)
    def _():
        out_ref[0, 0] = 0.0

    blk = r_ref[...]
    out_ref[0, 0] += jnp.sum(blk * blk)


def _sumsq(r):
    E = r.shape[0]
    return pl.pallas_call(
        _sumsq_body,
        grid=(E // _BE_SS,),
        in_specs=[pl.BlockSpec((_BE_SS, 3), lambda i: (i, 0))],
        out_specs=pl.BlockSpec((1, 1), lambda i: (0, 0),
                               memory_space=pltpu.SMEM),
        out_shape=jax.ShapeDtypeStruct((1, 1), jnp.float32),
    )(r)


_BE = 1280


def _msg_body(s_ref, r_ref, rt_ref, v_ref, wphi_ref, bphi_ref, ww_ref, bw_ref,
              ss_ref, out_ref):
    sblk = s_ref[...]
    phi = jnp.dot(sblk, wphi_ref[...],
                  preferred_element_type=jnp.float32) + bphi_ref[...]

    rblk = r_ref[...]
    rt = rt_ref[...]
    rn2t = jnp.sum(rt * rt, axis=0, keepdims=True)
    invt = lax.rsqrt(rn2t)
    rnt = rn2t * invt
    nvals = ((lax.broadcasted_iota(jnp.int32, (N_RBF, 1), 0) + 1)
             .astype(jnp.float32) * (math.pi / R_CUT))
    rbt = jnp.sin(rnt * nvals) * invt
    rbt = jnp.where(rbt <= R_CUT,
                    0.5 * (jnp.cos(rbt * (math.pi / R_CUT)) + 1.0),
                    0.0)
    w = lax.dot_general(rbt, ww_ref[...],
                        dimension_numbers=(((0,), (0,)), ((), ())),
                        preferred_element_type=jnp.float32) + bw_ref[...]

    split = w * phi
    s0 = split[:, :F]
    s1 = split[:, F:2 * F]
    s2 = split[:, 2 * F:]

    ginv = lax.rsqrt(ss_ref[0, 0])
    vblk = v_ref[...]
    out_ref[0] = s1
    for d in range(3):
        rd = rblk[:, d:d + 1] * ginv
        out_ref[d + 1] = s0 * vblk[:, d * F:(d + 1) * F] + s2 * rd


def _edge_messages(s2d, r, rt, v2d, W_phi, b_phi2, W_w, b_w2, ss):
    E = s2d.shape[0]
    return pl.pallas_call(
        _msg_body,
        grid=(E // _BE,),
        in_specs=[
            pl.BlockSpec((_BE, F), lambda i: (i, 0)),
            pl.BlockSpec((_BE, 3), lambda i: (i, 0)),
            pl.BlockSpec((3, _BE), lambda i: (0, i)),
            pl.BlockSpec((_BE, 3 * F), lambda i: (i, 0)),
            pl.BlockSpec((F, 3 * F), lambda i: (0, 0)),
            pl.BlockSpec((1, 3 * F), lambda i: (0, 0)),
            pl.BlockSpec((N_RBF, 3 * F), lambda i: (0, 0)),
            pl.BlockSpec((1, 3 * F), lambda i: (0, 0)),
            pl.BlockSpec((1, 1), lambda i: (0, 0), memory_space=pltpu.SMEM),
        ],
        out_specs=pl.BlockSpec((4, _BE, F), lambda i: (0, i, 0)),
        out_shape=jax.ShapeDtypeStruct((4, E, F), jnp.float32),
    )(s2d, r, rt, v2d, W_phi, b_phi2, W_w, b_w2, ss)


_C = 128
_CA = 80


def _sc_segment_sum(msg, idx3, zrows, E):
    n_tiles = 16
    ech = E // _C
    ach = N_NODES // _CA

    mesh = plsc.VectorSubcoreMesh(core_axis_name="c", subcore_axis_name="s")

    @functools.partial(
        pl.kernel,
        mesh=mesh,
        out_type=jax.ShapeDtypeStruct((4, N_NODES, F), jnp.float32),
        scratch_types=[
            pltpu.VMEM((_C,), jnp.int32),
            pltpu.VMEM((_C, F), jnp.float32),
            pltpu.VMEM((_CA, F), jnp.float32),
            pltpu.VMEM_SHARED((N_NODES, F), jnp.float32),
        ],
    )
    def run(msg_hbm, idx_hbm, zrows_hbm, out_hbm, idx_v, rows_v, zbuf_v, acc):
        core = lax.axis_index("c")
        sub = lax.axis_index("s")
        my_acc_chunks = (ach - sub + n_tiles - 1) // n_tiles
        my_edge_chunks = (ech - sub + n_tiles - 1) // n_tiles

        for p in range(2):
            plane = core * 2 + p

            pltpu.sync_copy(zrows_hbm, zbuf_v)

            def zbody(k, carry):
                ci = sub + n_tiles * k
                pltpu.sync_copy(zbuf_v, acc.at[pl.ds(ci * _CA, _CA)])
                return carry

            lax.fori_loop(0, my_acc_chunks, zbody, 0)
            plsc.subcore_barrier()

            def cbody(k, carry):
                ch = sub + n_tiles * k
                pltpu.sync_copy(idx_hbm.at[ch, 0], idx_v)
                pltpu.sync_copy(msg_hbm.at[plane, pl.ds(ch * _C, _C)], rows_v)
                pltpu.sync_copy(rows_v, acc.at[idx_v], add=True)
                return carry

            lax.fori_loop(0, my_edge_chunks, cbody, 0)
            plsc.subcore_barrier()

            def fbody(k, carry):
                ci = sub + n_tiles * k
                pltpu.sync_copy(acc.at[pl.ds(ci * _CA, _CA)],
                                out_hbm.at[plane, pl.ds(ci * _CA, _CA)])
                return carry

            lax.fori_loop(0, my_acc_chunks, fbody, 0)
            plsc.subcore_barrier()

    return run(msg, idx3, zrows)


_BN = 2000


def _unpack_body(acc_ref, outv_ref, outs_ref):
    outs_ref[:, 0, :] = acc_ref[0]
    for d in range(3):
        outv_ref[:, d, :] = acc_ref[d + 1]


def _unpack(acc4):
    return pl.pallas_call(
        _unpack_body,
        grid=(N_NODES // _BN,),
        in_specs=[pl.BlockSpec((4, _BN, F), lambda i: (0, i, 0))],
        out_specs=[
            pl.BlockSpec((_BN, 3, F), lambda i: (i, 0, 0)),
            pl.BlockSpec((_BN, 1, F), lambda i: (i, 0, 0)),
        ],
        out_shape=[
            jax.ShapeDtypeStruct((N_NODES, 3, F), jnp.float32),
            jax.ShapeDtypeStruct((N_NODES, 1, F), jnp.float32),
        ],
    )(acc4)


def kernel(s, r, v, W_phi, b_phi, W_w, b_w, idx_i):
    E = s.shape[0]
    s2d = s.reshape(E, F)
    v2d = v.reshape(E, 3 * F)
    idx = idx_i.astype(jnp.int32)
    b_phi2 = b_phi.reshape(1, 3 * F)
    b_w2 = b_w.reshape(1, 3 * F)

    rt = r.T
    ss = _sumsq(r)
    msg = _edge_messages(s2d, r, rt, v2d, W_phi, b_phi2, W_w, b_w2, ss)
    zrows = jnp.zeros((_CA, F), dtype=jnp.float32)
    idx = idx.reshape(E // _C, 1, _C)
    acc4 = _sc_segment_sum(msg, idx, zrows, E)
    out_v, out_s = _unpack(acc4)
    return (out_v, out_s)


# speedup vs baseline: 1.7138x; 1.5939x over previous
import functools
import math

import jax
import jax.numpy as jnp
from jax import lax
from jax.experimental import pallas as pl
from jax.experimental.pallas import tpu as pltpu
from jax.experimental.pallas import tpu_sc as plsc

F = 128
N_RBF = 20
R_CUT = 5.0
N_NODES = 10000


_BE_SS = 6400


def _sumsq_body(r_ref, out_ref):
    i = pl.program_id(0)

    @pl.when(i == 0)
    def _():
        out_ref[0, 0] = 0.0

    blk = r_ref[...]
    out_ref[0, 0] += jnp.sum(blk * blk)


def _sumsq(rt):
    E = rt.shape[1]
    return pl.pallas_call(
        _sumsq_body,
        grid=(E // _BE_SS,),
        in_specs=[pl.BlockSpec((3, _BE_SS), lambda i: (0, i))],
        out_specs=pl.BlockSpec((1, 1), lambda i: (0, 0),
                               memory_space=pltpu.SMEM),
        out_shape=jax.ShapeDtypeStruct((1, 1), jnp.float32),
    )(rt)


_BE = 1280


def _msg_body(s_ref, rt_ref, v_ref, wphi_ref, bphi_ref, ww_ref, bw_ref,
              ss_ref, out_ref):
    sblk = s_ref[:, 0, :]
    phi = jnp.dot(sblk, wphi_ref[...],
                  preferred_element_type=jnp.float32) + bphi_ref[...]

    rt = rt_ref[...]
    rn2t = jnp.sum(rt * rt, axis=0, keepdims=True)
    invt = lax.rsqrt(rn2t)
    rnt = rn2t * invt
    nvals = ((lax.broadcasted_iota(jnp.int32, (N_RBF, 1), 0) + 1)
             .astype(jnp.float32) * (math.pi / R_CUT))
    rbt = jnp.sin(rnt * nvals) * invt
    rbt = jnp.where(rbt <= R_CUT,
                    0.5 * (jnp.cos(rbt * (math.pi / R_CUT)) + 1.0),
                    0.0)
    w = lax.dot_general(rbt, ww_ref[...],
                        dimension_numbers=(((0,), (0,)), ((), ())),
                        preferred_element_type=jnp.float32) + bw_ref[...]

    split = w * phi
    s0 = split[:, :F]
    s1 = split[:, F:2 * F]
    s2 = split[:, 2 * F:]

    ginv = lax.rsqrt(ss_ref[0, 0])
    di = lax.broadcasted_iota(jnp.int32, (3, 3 * F), 0)
    dj = lax.broadcasted_iota(jnp.int32, (3, 3 * F), 1) // F
    sel = jnp.where(di == dj, ginv, 0.0)
    rdb = lax.dot_general(rt, sel,
                          dimension_numbers=(((0,), (0,)), ((), ())),
                          preferred_element_type=jnp.float32)
    out_ref[0] = s1
    for d in range(3):
        out_ref[d + 1] = (s0 * v_ref[d]
                          + s2 * rdb[:, d * F:(d + 1) * F])


def _edge_messages(s3, rt, vt, W_phi, b_phi2, W_w, b_w2, ss):
    E = s3.shape[0]
    return pl.pallas_call(
        _msg_body,
        grid=(E // _BE,),
        in_specs=[
            pl.BlockSpec((_BE, 1, F), lambda i: (i, 0, 0)),
            pl.BlockSpec((3, _BE), lambda i: (0, i)),
            pl.BlockSpec((3, _BE, F), lambda i: (0, i, 0)),
            pl.BlockSpec((F, 3 * F), lambda i: (0, 0)),
            pl.BlockSpec((1, 3 * F), lambda i: (0, 0)),
            pl.BlockSpec((N_RBF, 3 * F), lambda i: (0, 0)),
            pl.BlockSpec((1, 3 * F), lambda i: (0, 0)),
            pl.BlockSpec((1, 1), lambda i: (0, 0), memory_space=pltpu.SMEM),
        ],
        out_specs=pl.BlockSpec((4, _BE, F), lambda i: (0, i, 0)),
        out_shape=jax.ShapeDtypeStruct((4, E, F), jnp.float32),
    )(s3, rt, vt, W_phi, b_phi2, W_w, b_w2, ss)


_C = 128
_CA = 80


def _sc_segment_sum(msg, idx3, zrows, E):
    n_tiles = 16
    ech = E // _C
    ach = N_NODES // _CA

    mesh = plsc.VectorSubcoreMesh(core_axis_name="c", subcore_axis_name="s")

    @functools.partial(
        pl.kernel,
        mesh=mesh,
        out_type=jax.ShapeDtypeStruct((4, N_NODES, F), jnp.float32),
        scratch_types=[
            pltpu.VMEM((_C,), jnp.int32),
            pltpu.VMEM((_C, F), jnp.float32),
            pltpu.VMEM((_CA, F), jnp.float32),
            pltpu.VMEM_SHARED((N_NODES, F), jnp.float32),
        ],
    )
    def run(msg_hbm, idx_hbm, zrows_hbm, out_hbm, idx_v, rows_v, zbuf_v, acc):
        core = lax.axis_index("c")
        sub = lax.axis_index("s")
        my_acc_chunks = (ach - sub + n_tiles - 1) // n_tiles
        my_edge_chunks = (ech - sub + n_tiles - 1) // n_tiles

        for p in range(2):
            plane = core * 2 + p

            pltpu.sync_copy(zrows_hbm, zbuf_v)

            def zbody(k, carry):
                ci = sub + n_tiles * k
                pltpu.sync_copy(zbuf_v, acc.at[pl.ds(ci * _CA, _CA)])
                return carry

            lax.fori_loop(0, my_acc_chunks, zbody, 0)
            plsc.subcore_barrier()

            def cbody(k, carry):
                ch = sub + n_tiles * k
                pltpu.sync_copy(idx_hbm.at[ch, 0], idx_v)
                pltpu.sync_copy(msg_hbm.at[plane, pl.ds(ch * _C, _C)], rows_v)
                pltpu.sync_copy(rows_v, acc.at[idx_v], add=True)
                return carry

            lax.fori_loop(0, my_edge_chunks, cbody, 0)
            plsc.subcore_barrier()

            def fbody(k, carry):
                ci = sub + n_tiles * k
                pltpu.sync_copy(acc.at[pl.ds(ci * _CA, _CA)],
                                out_hbm.at[plane, pl.ds(ci * _CA, _CA)])
                return carry

            lax.fori_loop(0, my_acc_chunks, fbody, 0)
            plsc.subcore_barrier()

    return run(msg, idx3, zrows)


_BN = 2000


def _unpack_body(acc_ref, outv_ref, outs_ref):
    outs_ref[:, 0, :] = acc_ref[0]
    for d in range(3):
        outv_ref[d] = acc_ref[d + 1]


def _unpack(acc4):
    return pl.pallas_call(
        _unpack_body,
        grid=(N_NODES // _BN,),
        in_specs=[pl.BlockSpec((4, _BN, F), lambda i: (0, i, 0))],
        out_specs=[
            pl.BlockSpec((3, _BN, F), lambda i: (0, i, 0)),
            pl.BlockSpec((_BN, 1, F), lambda i: (i, 0, 0)),
        ],
        out_shape=[
            jax.ShapeDtypeStruct((3, N_NODES, F), jnp.float32),
            jax.ShapeDtypeStruct((N_NODES, 1, F), jnp.float32),
        ],
    )(acc4)


def kernel(s, r, v, W_phi, b_phi, W_w, b_w, idx_i):
    E = s.shape[0]
    idx = idx_i.astype(jnp.int32)
    b_phi2 = b_phi.reshape(1, 3 * F)
    b_w2 = b_w.reshape(1, 3 * F)

    rt = r.T
    vt = v.transpose(1, 0, 2)
    ss = _sumsq(rt)
    msg = _edge_messages(s, rt, vt, W_phi, b_phi2, W_w, b_w2, ss)
    zrows = jnp.zeros((_CA, F), dtype=jnp.float32)
    idx = idx.reshape(E // _C, 1, _C)
    acc4 = _sc_segment_sum(msg, idx, zrows, E)
    outv3, out_s = _unpack(acc4)
    out_v = outv3.transpose(1, 0, 2)
    return (out_v, out_s)


# speedup vs baseline: 2.3454x; 1.3685x over previous
import functools
import math

import jax
import jax.numpy as jnp
from jax import lax
from jax.experimental import pallas as pl
from jax.experimental.pallas import tpu as pltpu
from jax.experimental.pallas import tpu_sc as plsc

F = 128
N_RBF = 20
R_CUT = 5.0
N_NODES = 10000


_BE_SS = 6400


def _sumsq_body(r_ref, out_ref):
    i = pl.program_id(0)

    @pl.when(i == 0)
    def _():
        out_ref[0, 0] = 0.0

    blk = r_ref[...]
    out_ref[0, 0] += jnp.sum(blk * blk)


def _sumsq(rt):
    E = rt.shape[1]
    return pl.pallas_call(
        _sumsq_body,
        grid=(E // _BE_SS,),
        in_specs=[pl.BlockSpec((3, _BE_SS), lambda i: (0, i))],
        out_specs=pl.BlockSpec((1, 1), lambda i: (0, 0),
                               memory_space=pltpu.SMEM),
        out_shape=jax.ShapeDtypeStruct((1, 1), jnp.float32),
    )(rt)


_BE = 1280


def _msg_body(s_ref, rt_ref, v_ref, wphi_ref, bphi_ref, ww_ref, bw_ref,
              ss_ref, out_ref):
    sblk = s_ref[:, 0, :]
    phi = jnp.dot(sblk, wphi_ref[...],
                  preferred_element_type=jnp.float32) + bphi_ref[...]

    rt = rt_ref[...]
    rn2t = jnp.sum(rt * rt, axis=0, keepdims=True)
    invt = lax.rsqrt(rn2t)
    rnt = rn2t * invt
    nvals = ((lax.broadcasted_iota(jnp.int32, (N_RBF, 1), 0) + 1)
             .astype(jnp.float32) * (math.pi / R_CUT))
    rbt = jnp.sin(rnt * nvals) * invt
    rbt = jnp.where(rbt <= R_CUT,
                    0.5 * (jnp.cos(rbt * (math.pi / R_CUT)) + 1.0),
                    0.0)
    w = lax.dot_general(rbt, ww_ref[...],
                        dimension_numbers=(((0,), (0,)), ((), ())),
                        preferred_element_type=jnp.float32) + bw_ref[...]

    split = w * phi
    s0 = split[:, :F]
    s1 = split[:, F:2 * F]
    s2 = split[:, 2 * F:]

    ginv = lax.rsqrt(ss_ref[0, 0])
    di = lax.broadcasted_iota(jnp.int32, (3, 3 * F), 0)
    dj = lax.broadcasted_iota(jnp.int32, (3, 3 * F), 1) // F
    sel = jnp.where(di == dj, ginv, 0.0)
    rdb = lax.dot_general(rt, sel,
                          dimension_numbers=(((0,), (0,)), ((), ())),
                          preferred_element_type=jnp.float32)
    out_ref[0] = s1
    for d in range(3):
        out_ref[d + 1] = (s0 * v_ref[d]
                          + s2 * rdb[:, d * F:(d + 1) * F])


def _edge_messages(s3, rt, vt, W_phi, b_phi2, W_w, b_w2, ss):
    E = s3.shape[0]
    return pl.pallas_call(
        _msg_body,
        grid=(E // _BE,),
        in_specs=[
            pl.BlockSpec((_BE, 1, F), lambda i: (i, 0, 0)),
            pl.BlockSpec((3, _BE), lambda i: (0, i)),
            pl.BlockSpec((3, _BE, F), lambda i: (0, i, 0)),
            pl.BlockSpec((F, 3 * F), lambda i: (0, 0)),
            pl.BlockSpec((1, 3 * F), lambda i: (0, 0)),
            pl.BlockSpec((N_RBF, 3 * F), lambda i: (0, 0)),
            pl.BlockSpec((1, 3 * F), lambda i: (0, 0)),
            pl.BlockSpec((1, 1), lambda i: (0, 0), memory_space=pltpu.SMEM),
        ],
        out_specs=pl.BlockSpec((4, _BE, F), lambda i: (0, i, 0)),
        out_shape=jax.ShapeDtypeStruct((4, E, F), jnp.float32),
    )(s3, rt, vt, W_phi, b_phi2, W_w, b_w2, ss)


_C = 128
_CA = 80


def _sc_segment_sum(msg, idx3, zrows, E):
    n_tiles = 16
    ech = E // _C
    ach = N_NODES // _CA

    mesh = plsc.VectorSubcoreMesh(core_axis_name="c", subcore_axis_name="s")

    @functools.partial(
        pl.kernel,
        mesh=mesh,
        out_type=jax.ShapeDtypeStruct((4, N_NODES, F), jnp.float32),
        scratch_types=[
            pltpu.VMEM((_C,), jnp.int32),
            pltpu.VMEM((_C,), jnp.int32),
            pltpu.VMEM((_C, F), jnp.float32),
            pltpu.VMEM((_C, F), jnp.float32),
            pltpu.VMEM((_CA, F), jnp.float32),
            pltpu.VMEM_SHARED((N_NODES, F), jnp.float32),
            pltpu.SemaphoreType.DMA,
            pltpu.SemaphoreType.DMA,
        ],
    )
    def run(msg_hbm, idx_hbm, zrows_hbm, out_hbm, idx_v0, idx_v1, rows_v0,
            rows_v1, zbuf_v, acc, sem0, sem1):
        core = lax.axis_index("c")
        sub = lax.axis_index("s")
        my_acc_chunks = (ach - sub + n_tiles - 1) // n_tiles
        uni = ech // n_tiles
        tail = ech - uni * n_tiles

        for p in range(2):
            plane = core * 2 + p

            pltpu.sync_copy(zrows_hbm, zbuf_v)

            def zbody(k, carry):
                ci = sub + n_tiles * k
                pltpu.sync_copy(zbuf_v, acc.at[pl.ds(ci * _CA, _CA)])
                return carry

            lax.fori_loop(0, my_acc_chunks, zbody, 0)
            plsc.subcore_barrier()

            bufs = ((idx_v0, rows_v0, sem0), (idx_v1, rows_v1, sem1))

            def start(b, k):
                ch = sub + n_tiles * k
                iv, rv, sem = bufs[b]
                pltpu.async_copy(idx_hbm.at[ch, 0], iv, sem)
                pltpu.async_copy(msg_hbm.at[plane, pl.ds(ch * _C, _C)], rv,
                                 sem)

            def finish(b, k):
                ch = sub + n_tiles * k
                iv, rv, sem = bufs[b]
                pltpu.make_async_copy(idx_hbm.at[ch, 0], iv, sem).wait()
                pltpu.make_async_copy(
                    msg_hbm.at[plane, pl.ds(ch * _C, _C)], rv, sem).wait()
                pltpu.sync_copy(rv, acc.at[iv], add=True)

            npairs = uni // 2
            start(0, 0)

            def gbody(g, carry):
                k0 = 2 * g
                start(1, k0 + 1)
                finish(0, k0)

                @pl.when(g + 1 < npairs)
                def _():
                    start(0, k0 + 2)

                finish(1, k0 + 1)
                return carry

            lax.fori_loop(0, npairs, gbody, 0)

            @pl.when(sub < tail)
            def _():
                ch = uni * n_tiles + sub
                pltpu.sync_copy(idx_hbm.at[ch, 0], idx_v0)
                pltpu.sync_copy(msg_hbm.at[plane, pl.ds(ch * _C, _C)],
                                rows_v0)
                pltpu.sync_copy(rows_v0, acc.at[idx_v0], add=True)

            plsc.subcore_barrier()

            def fbody(k, carry):
                ci = sub + n_tiles * k
                pltpu.sync_copy(acc.at[pl.ds(ci * _CA, _CA)],
                                out_hbm.at[plane, pl.ds(ci * _CA, _CA)])
                return carry

            lax.fori_loop(0, my_acc_chunks, fbody, 0)
            plsc.subcore_barrier()

    return run(msg, idx3, zrows)


_BN = 2000


def _unpack_body(acc_ref, outv_ref, outs_ref):
    outs_ref[:, 0, :] = acc_ref[0]
    for d in range(3):
        outv_ref[d] = acc_ref[d + 1]


def _unpack(acc4):
    return pl.pallas_call(
        _unpack_body,
        grid=(N_NODES // _BN,),
        in_specs=[pl.BlockSpec((4, _BN, F), lambda i: (0, i, 0))],
        out_specs=[
            pl.BlockSpec((3, _BN, F), lambda i: (0, i, 0)),
            pl.BlockSpec((_BN, 1, F), lambda i: (i, 0, 0)),
        ],
        out_shape=[
            jax.ShapeDtypeStruct((3, N_NODES, F), jnp.float32),
            jax.ShapeDtypeStruct((N_NODES, 1, F), jnp.float32),
        ],
    )(acc4)


def kernel(s, r, v, W_phi, b_phi, W_w, b_w, idx_i):
    E = s.shape[0]
    idx = idx_i.astype(jnp.int32)
    b_phi2 = b_phi.reshape(1, 3 * F)
    b_w2 = b_w.reshape(1, 3 * F)

    rt = r.T
    vt = v.transpose(1, 0, 2)
    ss = _sumsq(rt)
    msg = _edge_messages(s, rt, vt, W_phi, b_phi2, W_w, b_w2, ss)
    zrows = jnp.zeros((_CA, F), dtype=jnp.float32)
    idx = idx.reshape(E // _C, 1, _C)
    acc4 = _sc_segment_sum(msg, idx, zrows, E)
    outv3, out_s = _unpack(acc4)
    out_v = outv3.transpose(1, 0, 2)
    return (out_v, out_s)


# speedup vs baseline: 2.5672x; 1.0945x over previous
import functools
import math

import jax
import jax.numpy as jnp
from jax import lax
from jax.experimental import pallas as pl
from jax.experimental.pallas import tpu as pltpu
from jax.experimental.pallas import tpu_sc as plsc

F = 128
N_RBF = 20
R_CUT = 5.0
N_NODES = 10000


_BE_SS = 6400


def _sumsq_body(r_ref, out_ref):
    i = pl.program_id(0)

    @pl.when(i == 0)
    def _():
        out_ref[0, 0] = 0.0

    blk = r_ref[...]
    out_ref[0, 0] += jnp.sum(blk * blk)


def _sumsq(rt):
    E = rt.shape[1]
    return pl.pallas_call(
        _sumsq_body,
        grid=(E // _BE_SS,),
        in_specs=[pl.BlockSpec((3, _BE_SS), lambda i: (0, i))],
        out_specs=pl.BlockSpec((1, 1), lambda i: (0, 0),
                               memory_space=pltpu.SMEM),
        out_shape=jax.ShapeDtypeStruct((1, 1), jnp.float32),
    )(rt)


_BE = 3200


def _msg_body(s_ref, rt_ref, v_ref, wphi_ref, bphi_ref, ww_ref, bw_ref,
              ss_ref, out_ref):
    sblk = s_ref[:, 0, :]
    phi = jnp.dot(sblk, wphi_ref[...],
                  preferred_element_type=jnp.float32) + bphi_ref[...]

    rt = rt_ref[...]
    rn2t = jnp.sum(rt * rt, axis=0, keepdims=True)
    invt = lax.rsqrt(rn2t)
    rnt = rn2t * invt
    nvals = ((lax.broadcasted_iota(jnp.int32, (N_RBF, 1), 0) + 1)
             .astype(jnp.float32) * (math.pi / R_CUT))
    rbt = jnp.sin(rnt * nvals) * invt
    rbt = jnp.where(rbt <= R_CUT,
                    0.5 * (jnp.cos(rbt * (math.pi / R_CUT)) + 1.0),
                    0.0)
    w = lax.dot_general(rbt, ww_ref[...],
                        dimension_numbers=(((0,), (0,)), ((), ())),
                        preferred_element_type=jnp.float32) + bw_ref[...]

    split = w * phi
    s0 = split[:, :F]
    s1 = split[:, F:2 * F]
    s2 = split[:, 2 * F:]

    ginv = lax.rsqrt(ss_ref[0, 0])
    di = lax.broadcasted_iota(jnp.int32, (3, 3 * F), 0)
    dj = lax.broadcasted_iota(jnp.int32, (3, 3 * F), 1) // F
    sel = jnp.where(di == dj, ginv, 0.0)
    rdb = lax.dot_general(rt, sel,
                          dimension_numbers=(((0,), (0,)), ((), ())),
                          preferred_element_type=jnp.float32)
    out_ref[0] = s1
    for d in range(3):
        out_ref[d + 1] = (s0 * v_ref[d]
                          + s2 * rdb[:, d * F:(d + 1) * F])


def _edge_messages(s3, rt, vt, W_phi, b_phi2, W_w, b_w2, ss):
    E = s3.shape[0]
    return pl.pallas_call(
        _msg_body,
        grid=(E // _BE,),
        in_specs=[
            pl.BlockSpec((_BE, 1, F), lambda i: (i, 0, 0)),
            pl.BlockSpec((3, _BE), lambda i: (0, i)),
            pl.BlockSpec((3, _BE, F), lambda i: (0, i, 0)),
            pl.BlockSpec((F, 3 * F), lambda i: (0, 0)),
            pl.BlockSpec((1, 3 * F), lambda i: (0, 0)),
            pl.BlockSpec((N_RBF, 3 * F), lambda i: (0, 0)),
            pl.BlockSpec((1, 3 * F), lambda i: (0, 0)),
            pl.BlockSpec((1, 1), lambda i: (0, 0), memory_space=pltpu.SMEM),
        ],
        out_specs=pl.BlockSpec((4, _BE, F), lambda i: (0, i, 0)),
        out_shape=jax.ShapeDtypeStruct((4, E, F), jnp.float32),
    )(s3, rt, vt, W_phi, b_phi2, W_w, b_w2, ss)


_C = 128
_CA = 80


def _sc_segment_sum(msg, idx3, zrows, E):
    n_tiles = 16
    ech = E // _C
    ach = N_NODES // _CA

    mesh = plsc.VectorSubcoreMesh(core_axis_name="c", subcore_axis_name="s")

    @functools.partial(
        pl.kernel,
        mesh=mesh,
        out_type=jax.ShapeDtypeStruct((4, N_NODES, F), jnp.float32),
        scratch_types=[
            pltpu.VMEM((_C,), jnp.int32),
            pltpu.VMEM((_C,), jnp.int32),
            pltpu.VMEM((_C, F), jnp.float32),
            pltpu.VMEM((_C, F), jnp.float32),
            pltpu.VMEM((_CA, F), jnp.float32),
            pltpu.VMEM_SHARED((N_NODES, F), jnp.float32),
            pltpu.SemaphoreType.DMA,
            pltpu.SemaphoreType.DMA,
        ],
    )
    def run(msg_hbm, idx_hbm, zrows_hbm, out_hbm, idx_v0, idx_v1, rows_v0,
            rows_v1, zbuf_v, acc, sem0, sem1):
        core = lax.axis_index("c")
        sub = lax.axis_index("s")
        my_acc_chunks = (ach - sub + n_tiles - 1) // n_tiles
        uni = ech // n_tiles
        tail = ech - uni * n_tiles

        for p in range(2):
            plane = core * 2 + p

            pltpu.sync_copy(zrows_hbm, zbuf_v)

            def zbody(k, carry):
                ci = sub + n_tiles * k
                pltpu.sync_copy(zbuf_v, acc.at[pl.ds(ci * _CA, _CA)])
                return carry

            lax.fori_loop(0, my_acc_chunks, zbody, 0)
            plsc.subcore_barrier()

            bufs = ((idx_v0, rows_v0, sem0), (idx_v1, rows_v1, sem1))

            def start(b, k):
                ch = sub + n_tiles * k
                iv, rv, sem = bufs[b]
                pltpu.async_copy(idx_hbm.at[ch, 0], iv, sem)
                pltpu.async_copy(msg_hbm.at[plane, pl.ds(ch * _C, _C)], rv,
                                 sem)

            def finish(b, k):
                ch = sub + n_tiles * k
                iv, rv, sem = bufs[b]
                pltpu.make_async_copy(idx_hbm.at[ch, 0], iv, sem).wait()
                pltpu.make_async_copy(
                    msg_hbm.at[plane, pl.ds(ch * _C, _C)], rv, sem).wait()
                pltpu.sync_copy(rv, acc.at[iv], add=True)

            npairs = uni // 2
            start(0, 0)

            def gbody(g, carry):
                k0 = 2 * g
                start(1, k0 + 1)
                finish(0, k0)

                @pl.when(g + 1 < npairs)
                def _():
                    start(0, k0 + 2)

                finish(1, k0 + 1)
                return carry

            lax.fori_loop(0, npairs, gbody, 0)

            @pl.when(sub < tail)
            def _():
                ch = uni * n_tiles + sub
                pltpu.sync_copy(idx_hbm.at[ch, 0], idx_v0)
                pltpu.sync_copy(msg_hbm.at[plane, pl.ds(ch * _C, _C)],
                                rows_v0)
                pltpu.sync_copy(rows_v0, acc.at[idx_v0], add=True)

            plsc.subcore_barrier()

            def fbody(k, carry):
                ci = sub + n_tiles * k
                pltpu.sync_copy(acc.at[pl.ds(ci * _CA, _CA)],
                                out_hbm.at[plane, pl.ds(ci * _CA, _CA)])
                return carry

            lax.fori_loop(0, my_acc_chunks, fbody, 0)
            plsc.subcore_barrier()

    return run(msg, idx3, zrows)


_BN = 2000


def _unpack_body(acc_ref, outv_ref, outs_ref):
    outs_ref[:, 0, :] = acc_ref[0]
    for d in range(3):
        outv_ref[d] = acc_ref[d + 1]


def _unpack(acc4):
    return pl.pallas_call(
        _unpack_body,
        grid=(N_NODES // _BN,),
        in_specs=[pl.BlockSpec((4, _BN, F), lambda i: (0, i, 0))],
        out_specs=[
            pl.BlockSpec((3, _BN, F), lambda i: (0, i, 0)),
            pl.BlockSpec((_BN, 1, F), lambda i: (i, 0, 0)),
        ],
        out_shape=[
            jax.ShapeDtypeStruct((3, N_NODES, F), jnp.float32),
            jax.ShapeDtypeStruct((N_NODES, 1, F), jnp.float32),
        ],
    )(acc4)


def kernel(s, r, v, W_phi, b_phi, W_w, b_w, idx_i):
    E = s.shape[0]
    idx = idx_i.astype(jnp.int32)
    b_phi2 = b_phi.reshape(1, 3 * F)
    b_w2 = b_w.reshape(1, 3 * F)

    rt = r.T
    vt = v.transpose(1, 0, 2)
    ss = _sumsq(rt)
    msg = _edge_messages(s, rt, vt, W_phi, b_phi2, W_w, b_w2, ss)
    zrows = jnp.zeros((_CA, F), dtype=jnp.float32)
    idx = idx.reshape(E // _C, 1, _C)
    acc4 = _sc_segment_sum(msg, idx, zrows, E)
    outv3, out_s = _unpack(acc4)
    out_v = outv3.transpose(1, 0, 2)
    return (out_v, out_s)
